# baseline (speedup 1.0000x reference)
import functools

import jax
import jax.numpy as jnp
from jax import lax
from jax.experimental import pallas as pl
from jax.experimental.pallas import tpu as pltpu
from jax.experimental.pallas import tpu_sc as plsc

_NC = 2
_NS = 16
_NW = _NC * _NS
_L = 16


def _replicate(src_ref, src_row, src_off, dst_ref, dst_off, rows, d4):
    vecs = [src_ref[src_row, pl.ds(src_off + k * _L, _L)]
            for k in range(d4 // _L)]

    def body(c, carry):
        for k in range(d4 // _L):
            dst_ref[c, pl.ds(dst_off + k * _L, _L)] = vecs[k]
        return carry

    lax.fori_loop(0, rows, body, 0)


def _sc_body(gd, ng, d4, row_hbm, col_hbm, io_hbm, pair_hbm, out_hbm,
             rowcol0, rowcol1, iop_even, iop_odd, io_s, pair_s,
             sem_rc, sem_iop):
    r_per_w = gd // _NW
    hg = gd // 2
    wid = lax.axis_index("s") * _NC + lax.axis_index("c")
    r0 = wid * r_per_w
    rowcols = [rowcol0, rowcol1]
    iops = [iop_even, iop_odd]

    pltpu.sync_copy(io_hbm, io_s)
    pltpu.sync_copy(pair_hbm, pair_s)
    for rl in range(r_per_w):
        pltpu.sync_copy(col_hbm, rowcols[rl].at[:, pl.ds(d4, d4)])
        pltpu.sync_copy(row_hbm.at[r0 + rl],
                        rowcols[rl].at[0, pl.ds(0, d4)])
        _replicate(rowcols[rl], 0, 0, rowcols[rl], 0, gd, d4)
    for par in range(2):
        _replicate(io_s, par, 0, iops[par], 0, hg, d4)

    def drain_rc():
        pltpu.make_async_copy(
            rowcol0, out_hbm.at[0, 0, :, pl.ds(0, 2 * d4)], sem_rc).wait()

    def drain_iop():
        pltpu.make_async_copy(
            iop_even, out_hbm.at[0, 0, pl.ds(0, hg), pl.ds(2 * d4, 2 * d4)],
            sem_iop).wait()

    rc_out = 0
    iop_out = 0
    for g in range(ng):
        if g % 2 == 0:
            for _ in range(iop_out):
                drain_iop()
            iop_out = 0
            for par in range(2):
                _replicate(pair_s, g // 2, 0, iops[par], d4, hg, d4)
        for rl in range(r_per_w):
            r = r0 + rl
            pltpu.async_copy(
                rowcols[rl], out_hbm.at[g, r, :, pl.ds(0, 2 * d4)], sem_rc)
            rc_out += 1
            for h in range(2):
                pltpu.async_copy(
                    iops[g % 2],
                    out_hbm.at[g, r, pl.ds(h * hg, hg),
                               pl.ds(2 * d4, 2 * d4)],
                    sem_iop)
                iop_out += 1
        while rc_out > 8:
            drain_rc()
            rc_out -= 1
    for _ in range(rc_out):
        drain_rc()
    for _ in range(iop_out):
        drain_iop()


def kernel(row_table, col_table, io_table, pair_table, num_grids, grid_dim):
    gd = row_table.shape[0]
    ng = pair_table.shape[0] - 1
    d4 = row_table.shape[-1]
    d = 4 * d4

    mesh = plsc.VectorSubcoreMesh(core_axis_name="c", subcore_axis_name="s")
    sc_fn = pl.kernel(
        functools.partial(_sc_body, gd, ng, d4),
        mesh=mesh,
        out_type=jax.ShapeDtypeStruct((ng, gd, gd, d), row_table.dtype),
        scratch_types=[
            pltpu.VMEM((gd, 2 * d4), jnp.float32),
            pltpu.VMEM((gd, 2 * d4), jnp.float32),
            pltpu.VMEM((gd // 2, 2 * d4), jnp.float32),
            pltpu.VMEM((gd // 2, 2 * d4), jnp.float32),
            pltpu.VMEM(io_table.shape, jnp.float32),
            pltpu.VMEM(pair_table.shape, jnp.float32),
            pltpu.SemaphoreType.DMA,
            pltpu.SemaphoreType.DMA,
        ],
    )
    return sc_fn(row_table, col_table, io_table, pair_table)
